# baseline (speedup 1.0000x reference)
import jax
import jax.numpy as jnp
from jax import lax
from jax.experimental import pallas as pl
from jax.experimental.pallas import tpu as pltpu
from jax.experimental.pallas import tpu_sc as plsc

B = 4096
L = 50
EMB = 64
FEAT = 64
LINE = 128
VOCAB1 = 1000001
NW = 32
NB = 2 * B
ROWS = NB * L
STRIPE = NB // NW
CHUNK = 128
CPS = STRIPE // CHUNK
TBLK = 4096
NTB = (VOCAB1 + TBLK - 1) // TBLK
VPAD = NTB * TBLK


def _repack_body(embt_ref, out_ref):
    xt = jnp.transpose(embt_ref[...], (1, 0))
    out_ref[...] = jnp.concatenate(
        [xt, jnp.zeros((TBLK, LINE - EMB), jnp.float32)], axis=1)


def _repack(embt):
    return pl.pallas_call(
        _repack_body,
        grid=(NTB,),
        in_specs=[pl.BlockSpec((EMB, TBLK), lambda g: (0, g))],
        out_specs=pl.BlockSpec((TBLK, LINE), lambda g: (g, 0)),
        out_shape=jax.ShapeDtypeStruct((VPAD, LINE), jnp.float32),
    )(embt)


def _sc_gather_body(f1_hbm, f2_hbm, table_hbm, x_hbm,
                    idx_v, idxt_v, rows_v, gsem, ssem0, ssem1):
    c = lax.axis_index("c")
    s = lax.axis_index("s")
    w = s * 2 + c
    base = w * STRIPE

    @pl.when(w < NW // 2)
    def _():
        pltpu.sync_copy(f1_hbm.at[pl.ds(base, STRIPE)], idx_v)

    @pl.when(w >= NW // 2)
    def _():
        pltpu.sync_copy(f2_hbm.at[pl.ds(base - B, STRIPE)], idx_v)

    lanes = lax.iota(jnp.int32, 16)

    def tbody(t, carry):
        col = jnp.full((16,), t, dtype=jnp.int32)
        for j in range(STRIPE // 16):
            v = plsc.load_gather(idx_v, [j * 16 + lanes, col])
            idxt_v[t, pl.ds(j * 16, 16)] = v
        return carry

    lax.fori_loop(0, L, tbody, 0)

    def drain(buf, sem):
        pltpu.make_async_copy(
            rows_v.at[buf], x_hbm.at[pl.ds(0, STRIPE)], sem
        ).wait()

    sems = (ssem0, ssem1)
    lane64 = jnp.full((16,), EMB, dtype=jnp.int32)

    def gbody(g, carry):
        for buf in range(2):
            t = g * 2 + buf
            sem = sems[buf]

            @pl.when(g >= 1)
            def _():
                drain(buf, sem)

            descs = []
            for j in range(CPS):
                d = pltpu.async_copy(
                    table_hbm.at[idxt_v.at[t, pl.ds(j * CHUNK, CHUNK)]],
                    rows_v.at[buf, pl.ds(j * CHUNK, CHUNK)],
                    gsem,
                )
                descs.append(d)
            for d in descs:
                d.wait()
            for j in range(STRIPE // 16):
                v = idxt_v[t, pl.ds(j * 16, 16)]
                m = jnp.where(v != 0, 1.0, 0.0).astype(jnp.float32)
                plsc.store_scatter(
                    rows_v.at[buf], [j * 16 + lanes, lane64], m)
            pltpu.async_copy(
                rows_v.at[buf],
                x_hbm.at[pl.ds(t * NB + base, STRIPE)],
                sem,
            )
        return carry

    lax.fori_loop(0, L // 2, gbody, 0)
    drain(0, ssem0)
    drain(1, ssem1)


def _sc_gather(f1p, f2p, table):
    mesh = plsc.VectorSubcoreMesh(core_axis_name="c", subcore_axis_name="s")
    f = pl.kernel(
        _sc_gather_body,
        out_type=jax.ShapeDtypeStruct((ROWS, LINE), jnp.float32),
        mesh=mesh,
        scratch_types=[
            pltpu.VMEM((STRIPE, LINE), jnp.int32),
            pltpu.VMEM((L, STRIPE), jnp.int32),
            pltpu.VMEM((2, STRIPE, LINE), jnp.float32),
            pltpu.SemaphoreType.DMA,
            pltpu.SemaphoreType.DMA,
            pltpu.SemaphoreType.DMA,
        ],
        compiler_params=pltpu.CompilerParams(
            use_tc_tiling_on_sc=False, needs_layout_passes=False
        ),
    )
    return f(f1p, f2p, table)


def _tc_rnn_body(x_ref, w_ref, u_ref, b_ref, s1_ref, s2_ref, sim_ref, h_s):
    t = pl.program_id(0)

    @pl.when(t == 0)
    def _():
        h_s[...] = jnp.zeros_like(h_s)

    h = h_s[...]
    x = x_ref[0]
    xw = jnp.dot(x, w_ref[...], preferred_element_type=jnp.float32)
    hu = jnp.dot(h, u_ref[...], preferred_element_type=jnp.float32)
    h_new = jnp.tanh(xw + hu + b_ref[...])
    m = x[:, EMB:EMB + 1] != 0.0
    h = jnp.where(m, h_new, h)
    h_s[...] = h

    @pl.when(t == L - 1)
    def _():
        s1 = h[:B]
        s2 = h[B:]
        n1 = jnp.sqrt(jnp.sum(s1 * s1, axis=1, keepdims=True)) + 1e-12
        n2 = jnp.sqrt(jnp.sum(s2 * s2, axis=1, keepdims=True)) + 1e-12
        s1_ref[...] = s1
        s2_ref[...] = s2
        sim_ref[...] = jnp.sum(s1 * s2, axis=1, keepdims=True) / (n1 * n2)


def _tc_rnn(x, Wp, U, b):
    return pl.pallas_call(
        _tc_rnn_body,
        grid=(L,),
        in_specs=[
            pl.BlockSpec((1, NB, LINE), lambda t: (t, 0, 0)),
            pl.BlockSpec((LINE, FEAT), lambda t: (0, 0)),
            pl.BlockSpec((FEAT, FEAT), lambda t: (0, 0)),
            pl.BlockSpec((1, FEAT), lambda t: (0, 0)),
        ],
        out_specs=[
            pl.BlockSpec((B, FEAT), lambda t: (0, 0)),
            pl.BlockSpec((B, FEAT), lambda t: (0, 0)),
            pl.BlockSpec((B, 1), lambda t: (0, 0)),
        ],
        out_shape=[
            jax.ShapeDtypeStruct((B, FEAT), jnp.float32),
            jax.ShapeDtypeStruct((B, FEAT), jnp.float32),
            jax.ShapeDtypeStruct((B, 1), jnp.float32),
        ],
        scratch_shapes=[pltpu.VMEM((NB, FEAT), jnp.float32)],
    )(x, Wp, U, b)


@jax.jit
def kernel(funcname_1, funcname_2, emb_table, W, U, b):
    embt = jnp.transpose(emb_table)
    table = _repack(embt)
    f1p = jnp.pad(funcname_1, ((0, 0), (0, LINE - L)))
    f2p = jnp.pad(funcname_2, ((0, 0), (0, LINE - L)))
    Wp = jnp.concatenate([W, jnp.zeros((LINE - EMB, FEAT), W.dtype)], axis=0)
    x = _sc_gather(f1p, f2p, table)
    x = x.reshape(L, NB, LINE)
    s1, s2, sim = _tc_rnn(x, Wp, U, b.reshape(1, FEAT))
    return (s1, s2, sim.reshape(B))


# speedup vs baseline: 1.0780x; 1.0780x over previous
import jax
import jax.numpy as jnp
from jax import lax
from jax.experimental import pallas as pl
from jax.experimental.pallas import tpu as pltpu
from jax.experimental.pallas import tpu_sc as plsc

B = 4096
L = 50
EMB = 64
FEAT = 64
LINE = 128
VOCAB1 = 1000001
NW = 32
NB = 2 * B
ROWS = NB * L
STRIPE = NB // NW
CHUNK = 128
CPS = STRIPE // CHUNK
TBLK = 2048
NTB = 245
HHALF = NTB * TBLK
VPAD = 2 * HHALF


def _repack_body(top_ref, bot_ref, out_ref):
    xt = jnp.transpose(top_ref[...], (1, 0))
    xb = jnp.transpose(bot_ref[...], (1, 0))
    out_ref[...] = jnp.concatenate([xt, xb], axis=1)


def _repack(embt):
    return pl.pallas_call(
        _repack_body,
        grid=(NTB,),
        in_specs=[
            pl.BlockSpec((EMB, TBLK), lambda g: (0, g)),
            pl.BlockSpec(
                (EMB, TBLK),
                lambda g: (0, jnp.minimum(g + NTB, (VOCAB1 - 1) // TBLK)),
            ),
        ],
        out_specs=pl.BlockSpec((TBLK, 2 * EMB), lambda g: (g, 0)),
        out_shape=jax.ShapeDtypeStruct((HHALF, 2 * EMB), jnp.float32),
    )(embt, embt)


def _sc_gather_body(f1_hbm, f2_hbm, table_hbm, x_hbm,
                    idx_v, idxt_v, tmp_v, tail_v, gsem, ssem0, ssem1):
    c = lax.axis_index("c")
    s = lax.axis_index("s")
    w = s * 2 + c
    base = w * STRIPE

    @pl.when(w < NW // 2)
    def _():
        pltpu.sync_copy(f1_hbm.at[pl.ds(base, STRIPE)], idx_v)

    @pl.when(w >= NW // 2)
    def _():
        pltpu.sync_copy(f2_hbm.at[pl.ds(base - B, STRIPE)], idx_v)

    zeros16 = jnp.zeros((16,), jnp.float32)

    def zbody(r, carry):
        for k in range(4):
            tail_v[0, r, pl.ds(k * 16, 16)] = zeros16
            tail_v[1, r, pl.ds(k * 16, 16)] = zeros16
        return carry

    lax.fori_loop(0, STRIPE, zbody, 0)

    lanes = lax.iota(jnp.int32, 16)

    def tbody(t, carry):
        col = jnp.full((16,), t, dtype=jnp.int32)
        for j in range(STRIPE // 16):
            v = plsc.load_gather(idx_v, [j * 16 + lanes, col])
            ge = (v >= HHALF).astype(jnp.int32)
            v2 = 2 * v - (2 * HHALF) * ge + ge
            idxt_v[t, pl.ds(j * 16, 16)] = v2
        return carry

    lax.fori_loop(0, L, tbody, 0)

    def drain_pair(buf, sem):
        pltpu.make_async_copy(
            tmp_v.at[buf],
            x_hbm.at[pl.ds(0, STRIPE), pl.ds(0, EMB)],
            sem,
        ).wait()
        pltpu.make_async_copy(
            tmp_v.at[buf],
            x_hbm.at[pl.ds(0, STRIPE), pl.ds(0, EMB)],
            sem,
        ).wait()

    sems = (ssem0, ssem1)
    col0 = jnp.full((16,), 0, dtype=jnp.int32)

    def gbody(g, carry):
        for buf in range(2):
            t = g * 2 + buf
            sem = sems[buf]

            @pl.when(g >= 1)
            def _():
                drain_pair(buf, sem)

            descs = []
            for j in range(CPS):
                d = pltpu.async_copy(
                    table_hbm.at[idxt_v.at[t, pl.ds(j * CHUNK, CHUNK)]],
                    tmp_v.at[buf, pl.ds(j * CHUNK, CHUNK)],
                    gsem,
                )
                descs.append(d)
            for d in descs:
                d.wait()
            for j in range(STRIPE // 16):
                v = idxt_v[t, pl.ds(j * 16, 16)]
                m = jnp.where(v != 0, 1.0, 0.0).astype(jnp.float32)
                plsc.store_scatter(
                    tail_v.at[buf], [j * 16 + lanes, col0], m)
            line0 = t * NB + base
            pltpu.async_copy(
                tmp_v.at[buf],
                x_hbm.at[pl.ds(line0, STRIPE), pl.ds(0, EMB)],
                sem,
            )
            pltpu.async_copy(
                tail_v.at[buf],
                x_hbm.at[pl.ds(line0, STRIPE), pl.ds(EMB, LINE - EMB)],
                sem,
            )
        return carry

    lax.fori_loop(0, L // 2, gbody, 0)
    drain_pair(0, ssem0)
    drain_pair(1, ssem1)


def _sc_gather(f1p, f2p, table):
    mesh = plsc.VectorSubcoreMesh(core_axis_name="c", subcore_axis_name="s")
    f = pl.kernel(
        _sc_gather_body,
        out_type=jax.ShapeDtypeStruct((ROWS, LINE), jnp.float32),
        mesh=mesh,
        scratch_types=[
            pltpu.VMEM((STRIPE, LINE), jnp.int32),
            pltpu.VMEM((L, STRIPE), jnp.int32),
            pltpu.VMEM((2, STRIPE, EMB), jnp.float32),
            pltpu.VMEM((2, STRIPE, LINE - EMB), jnp.float32),
            pltpu.SemaphoreType.DMA,
            pltpu.SemaphoreType.DMA,
            pltpu.SemaphoreType.DMA,
        ],
        compiler_params=pltpu.CompilerParams(
            use_tc_tiling_on_sc=False, needs_layout_passes=False
        ),
    )
    return f(f1p, f2p, table)


def _tc_rnn_body(x_ref, w_ref, u_ref, b_ref, s1_ref, s2_ref, sim_ref, h_s):
    t = pl.program_id(0)

    @pl.when(t == 0)
    def _():
        h_s[...] = jnp.zeros_like(h_s)

    h = h_s[...]
    x = x_ref[0]
    xw = jnp.dot(x, w_ref[...], preferred_element_type=jnp.float32)
    hu = jnp.dot(h, u_ref[...], preferred_element_type=jnp.float32)
    h_new = jnp.tanh(xw + hu + b_ref[...])
    m = x[:, EMB:EMB + 1] != 0.0
    h = jnp.where(m, h_new, h)
    h_s[...] = h

    @pl.when(t == L - 1)
    def _():
        s1 = h[:B]
        s2 = h[B:]
        n1 = jnp.sqrt(jnp.sum(s1 * s1, axis=1, keepdims=True)) + 1e-12
        n2 = jnp.sqrt(jnp.sum(s2 * s2, axis=1, keepdims=True)) + 1e-12
        s1_ref[...] = s1
        s2_ref[...] = s2
        sim_ref[...] = jnp.sum(s1 * s2, axis=1, keepdims=True) / (n1 * n2)


def _tc_rnn(x, Wp, U, b):
    return pl.pallas_call(
        _tc_rnn_body,
        grid=(L,),
        in_specs=[
            pl.BlockSpec((1, NB, LINE), lambda t: (t, 0, 0)),
            pl.BlockSpec((LINE, FEAT), lambda t: (0, 0)),
            pl.BlockSpec((FEAT, FEAT), lambda t: (0, 0)),
            pl.BlockSpec((1, FEAT), lambda t: (0, 0)),
        ],
        out_specs=[
            pl.BlockSpec((B, FEAT), lambda t: (0, 0)),
            pl.BlockSpec((B, FEAT), lambda t: (0, 0)),
            pl.BlockSpec((B, 1), lambda t: (0, 0)),
        ],
        out_shape=[
            jax.ShapeDtypeStruct((B, FEAT), jnp.float32),
            jax.ShapeDtypeStruct((B, FEAT), jnp.float32),
            jax.ShapeDtypeStruct((B, 1), jnp.float32),
        ],
        scratch_shapes=[pltpu.VMEM((NB, FEAT), jnp.float32)],
    )(x, Wp, U, b)


@jax.jit
def kernel(funcname_1, funcname_2, emb_table, W, U, b):
    embt = jnp.transpose(emb_table)
    table = _repack(embt).reshape(VPAD, EMB)
    f1p = jnp.pad(funcname_1, ((0, 0), (0, LINE - L)))
    f2p = jnp.pad(funcname_2, ((0, 0), (0, LINE - L)))
    Wp = jnp.concatenate([W, jnp.zeros((LINE - EMB, FEAT), W.dtype)], axis=0)
    x = _sc_gather(f1p, f2p, table)
    x = x.reshape(L, NB, LINE)
    s1, s2, sim = _tc_rnn(x, Wp, U, b.reshape(1, FEAT))
    return (s1, s2, sim.reshape(B))


# speedup vs baseline: 1.3044x; 1.2100x over previous
import jax
import jax.numpy as jnp
from jax import lax
from jax.experimental import pallas as pl
from jax.experimental.pallas import tpu as pltpu
from jax.experimental.pallas import tpu_sc as plsc

B = 4096
L = 50
EMB = 64
FEAT = 64
LINE = 128
VOCAB1 = 1000001
NW = 32
NB = 2 * B
ROWS = NB * L
STRIPE = NB // NW
CHUNK = 128
CPS = STRIPE // CHUNK
TBLK = 8192
NTB = 62
HHALF = NTB * TBLK
VPAD = 2 * HHALF


def _repack_body(top_ref, bot_ref, out_ref):
    xt = jnp.transpose(top_ref[...], (1, 0))
    xb = jnp.transpose(bot_ref[...], (1, 0))
    out_ref[...] = jnp.concatenate([xt, xb], axis=1)


def _repack(embt):
    return pl.pallas_call(
        _repack_body,
        grid=(NTB,),
        in_specs=[
            pl.BlockSpec((EMB, TBLK), lambda g: (0, g)),
            pl.BlockSpec(
                (EMB, TBLK),
                lambda g: (0, jnp.minimum(g + NTB, (VOCAB1 - 1) // TBLK)),
            ),
        ],
        out_specs=pl.BlockSpec((TBLK, 2 * EMB), lambda g: (g, 0)),
        out_shape=jax.ShapeDtypeStruct((HHALF, 2 * EMB), jnp.float32),
    )(embt, embt)


def _sc_gather_body(f1_hbm, f2_hbm, table_hbm, x_hbm,
                    idx_v, idxt_v, tmp_v, tail_v, gsem, ssem0, ssem1):
    c = lax.axis_index("c")
    s = lax.axis_index("s")
    w = s * 2 + c
    base = w * STRIPE

    @pl.when(w < NW // 2)
    def _():
        pltpu.sync_copy(f1_hbm.at[pl.ds(base, STRIPE)], idx_v)

    @pl.when(w >= NW // 2)
    def _():
        pltpu.sync_copy(f2_hbm.at[pl.ds(base - B, STRIPE)], idx_v)

    zeros16 = jnp.zeros((16,), jnp.float32)

    def zbody(r, carry):
        for k in range(4):
            tail_v[0, r, pl.ds(k * 16, 16)] = zeros16
            tail_v[1, r, pl.ds(k * 16, 16)] = zeros16
        return carry

    lax.fori_loop(0, STRIPE, zbody, 0)

    lanes = lax.iota(jnp.int32, 16)

    def tbody(t, carry):
        col = jnp.full((16,), t, dtype=jnp.int32)
        for j in range(STRIPE // 16):
            v = plsc.load_gather(idx_v, [j * 16 + lanes, col])
            ge = (v >= HHALF).astype(jnp.int32)
            v2 = 2 * v - (2 * HHALF) * ge + ge
            idxt_v[t, pl.ds(j * 16, 16)] = v2
        return carry

    lax.fori_loop(0, L, tbody, 0)

    def drain_pair(buf, sem):
        pltpu.make_async_copy(
            tmp_v.at[buf],
            x_hbm.at[pl.ds(0, STRIPE), pl.ds(0, EMB)],
            sem,
        ).wait()
        pltpu.make_async_copy(
            tmp_v.at[buf],
            x_hbm.at[pl.ds(0, STRIPE), pl.ds(0, EMB)],
            sem,
        ).wait()

    sems = (ssem0, ssem1)
    col0 = jnp.full((16,), 0, dtype=jnp.int32)

    def gbody(g, carry):
        for buf in range(2):
            t = g * 2 + buf
            sem = sems[buf]

            @pl.when(g >= 1)
            def _():
                drain_pair(buf, sem)

            descs = []
            for j in range(CPS):
                d = pltpu.async_copy(
                    table_hbm.at[idxt_v.at[t, pl.ds(j * CHUNK, CHUNK)]],
                    tmp_v.at[buf, pl.ds(j * CHUNK, CHUNK)],
                    gsem,
                )
                descs.append(d)
            for d in descs:
                d.wait()
            for j in range(STRIPE // 16):
                v = idxt_v[t, pl.ds(j * 16, 16)]
                m = jnp.where(v != 0, 1.0, 0.0).astype(jnp.float32)
                plsc.store_scatter(
                    tail_v.at[buf], [j * 16 + lanes, col0], m)
            line0 = t * NB + base
            pltpu.async_copy(
                tmp_v.at[buf],
                x_hbm.at[pl.ds(line0, STRIPE), pl.ds(0, EMB)],
                sem,
            )
            pltpu.async_copy(
                tail_v.at[buf],
                x_hbm.at[pl.ds(line0, STRIPE), pl.ds(EMB, LINE - EMB)],
                sem,
            )
        return carry

    lax.fori_loop(0, L // 2, gbody, 0)
    drain_pair(0, ssem0)
    drain_pair(1, ssem1)


def _sc_gather(f1p, f2p, table):
    mesh = plsc.VectorSubcoreMesh(core_axis_name="c", subcore_axis_name="s")
    f = pl.kernel(
        _sc_gather_body,
        out_type=jax.ShapeDtypeStruct((ROWS, LINE), jnp.float32),
        mesh=mesh,
        scratch_types=[
            pltpu.VMEM((STRIPE, LINE), jnp.int32),
            pltpu.VMEM((L, STRIPE), jnp.int32),
            pltpu.VMEM((2, STRIPE, EMB), jnp.float32),
            pltpu.VMEM((2, STRIPE, LINE - EMB), jnp.float32),
            pltpu.SemaphoreType.DMA,
            pltpu.SemaphoreType.DMA,
            pltpu.SemaphoreType.DMA,
        ],
        compiler_params=pltpu.CompilerParams(
            use_tc_tiling_on_sc=False, needs_layout_passes=False
        ),
    )
    return f(f1p, f2p, table)


def _tc_rnn_body(x_ref, w_ref, u_ref, b_ref, s1_ref, s2_ref, sim_ref, h_s):
    t = pl.program_id(0)

    @pl.when(t == 0)
    def _():
        h_s[...] = jnp.zeros_like(h_s)

    h = h_s[...]
    x = x_ref[0]
    xw = jnp.dot(x, w_ref[...], preferred_element_type=jnp.float32)
    hu = jnp.dot(h, u_ref[...], preferred_element_type=jnp.float32)
    h_new = jnp.tanh(xw + hu + b_ref[...])
    m = x[:, EMB:EMB + 1] != 0.0
    h = jnp.where(m, h_new, h)
    h_s[...] = h

    @pl.when(t == L - 1)
    def _():
        s1 = h[:B]
        s2 = h[B:]
        n1 = jnp.sqrt(jnp.sum(s1 * s1, axis=1, keepdims=True)) + 1e-12
        n2 = jnp.sqrt(jnp.sum(s2 * s2, axis=1, keepdims=True)) + 1e-12
        s1_ref[...] = s1
        s2_ref[...] = s2
        sim_ref[...] = jnp.sum(s1 * s2, axis=1, keepdims=True) / (n1 * n2)


def _tc_rnn(x, Wp, U, b):
    return pl.pallas_call(
        _tc_rnn_body,
        grid=(L,),
        in_specs=[
            pl.BlockSpec((1, NB, LINE), lambda t: (t, 0, 0)),
            pl.BlockSpec((LINE, FEAT), lambda t: (0, 0)),
            pl.BlockSpec((FEAT, FEAT), lambda t: (0, 0)),
            pl.BlockSpec((1, FEAT), lambda t: (0, 0)),
        ],
        out_specs=[
            pl.BlockSpec((B, FEAT), lambda t: (0, 0)),
            pl.BlockSpec((B, FEAT), lambda t: (0, 0)),
            pl.BlockSpec((B, 1), lambda t: (0, 0)),
        ],
        out_shape=[
            jax.ShapeDtypeStruct((B, FEAT), jnp.float32),
            jax.ShapeDtypeStruct((B, FEAT), jnp.float32),
            jax.ShapeDtypeStruct((B, 1), jnp.float32),
        ],
        scratch_shapes=[pltpu.VMEM((NB, FEAT), jnp.float32)],
    )(x, Wp, U, b)


@jax.jit
def kernel(funcname_1, funcname_2, emb_table, W, U, b):
    embt = jnp.transpose(emb_table)
    table = _repack(embt).reshape(VPAD, EMB)
    f1p = jnp.pad(funcname_1, ((0, 0), (0, LINE - L)))
    f2p = jnp.pad(funcname_2, ((0, 0), (0, LINE - L)))
    Wp = jnp.concatenate([W, jnp.zeros((LINE - EMB, FEAT), W.dtype)], axis=0)
    x = _sc_gather(f1p, f2p, table)
    x = x.reshape(L, NB, LINE)
    s1, s2, sim = _tc_rnn(x, Wp, U, b.reshape(1, FEAT))
    return (s1, s2, sim.reshape(B))


# speedup vs baseline: 1.3461x; 1.0320x over previous
import jax
import jax.numpy as jnp
from jax import lax
from jax.experimental import pallas as pl
from jax.experimental.pallas import tpu as pltpu
from jax.experimental.pallas import tpu_sc as plsc

B = 4096
L = 50
EMB = 64
FEAT = 64
LINE = 128
VOCAB1 = 1000001
NW = 32
NB = 2 * B
ROWS = NB * L
STRIPE = NB // NW
CHUNK = 128
CPS = STRIPE // CHUNK
TBLK = 16384
NTB = 31
HHALF = NTB * TBLK
VPAD = 2 * HHALF


def _repack_body(top_ref, bot_ref, out_ref):
    xt = jnp.transpose(top_ref[...], (1, 0))
    xb = jnp.transpose(bot_ref[...], (1, 0))
    out_ref[...] = jnp.concatenate([xt, xb], axis=1)


def _repack(embt):
    return pl.pallas_call(
        _repack_body,
        grid=(NTB,),
        in_specs=[
            pl.BlockSpec((EMB, TBLK), lambda g: (0, g)),
            pl.BlockSpec(
                (EMB, TBLK),
                lambda g: (0, jnp.minimum(g + NTB, (VOCAB1 - 1) // TBLK)),
            ),
        ],
        out_specs=pl.BlockSpec((TBLK, 2 * EMB), lambda g: (g, 0)),
        out_shape=jax.ShapeDtypeStruct((HHALF, 2 * EMB), jnp.float32),
    )(embt, embt)


def _sc_gather_body(f1_hbm, f2_hbm, table_hbm, x_hbm,
                    idx_v, idxt_v, tmp_v, tail_v, gsem, ssem0, ssem1):
    c = lax.axis_index("c")
    s = lax.axis_index("s")
    w = s * 2 + c
    base = w * STRIPE

    @pl.when(w < NW // 2)
    def _():
        pltpu.sync_copy(f1_hbm.at[pl.ds(base, STRIPE)], idx_v)

    @pl.when(w >= NW // 2)
    def _():
        pltpu.sync_copy(f2_hbm.at[pl.ds(base - B, STRIPE)], idx_v)

    zeros16 = jnp.zeros((16,), jnp.float32)

    def zbody(r, carry):
        for k in range(4):
            tail_v[0, r, pl.ds(k * 16, 16)] = zeros16
            tail_v[1, r, pl.ds(k * 16, 16)] = zeros16
        return carry

    lax.fori_loop(0, STRIPE, zbody, 0)

    lanes = lax.iota(jnp.int32, 16)

    def tbody(t, carry):
        col = jnp.full((16,), t, dtype=jnp.int32)
        for j in range(STRIPE // 16):
            v = plsc.load_gather(idx_v, [j * 16 + lanes, col])
            ge = (v >= HHALF).astype(jnp.int32)
            v2 = 2 * v - (2 * HHALF) * ge + ge
            idxt_v[t, pl.ds(j * 16, 16)] = v2
        return carry

    lax.fori_loop(0, L, tbody, 0)

    def drain_pair(buf, sem):
        pltpu.make_async_copy(
            tmp_v.at[buf],
            x_hbm.at[pl.ds(0, STRIPE), pl.ds(0, EMB)],
            sem,
        ).wait()
        pltpu.make_async_copy(
            tmp_v.at[buf],
            x_hbm.at[pl.ds(0, STRIPE), pl.ds(0, EMB)],
            sem,
        ).wait()

    sems = (ssem0, ssem1)
    col0 = jnp.full((16,), 0, dtype=jnp.int32)

    def gbody(g, carry):
        for buf in range(2):
            t = g * 2 + buf
            sem = sems[buf]

            @pl.when(g >= 1)
            def _():
                drain_pair(buf, sem)

            descs = []
            for j in range(CPS):
                d = pltpu.async_copy(
                    table_hbm.at[idxt_v.at[t, pl.ds(j * CHUNK, CHUNK)]],
                    tmp_v.at[buf, pl.ds(j * CHUNK, CHUNK)],
                    gsem,
                )
                descs.append(d)
            for d in descs:
                d.wait()
            for j in range(STRIPE // 16):
                v = idxt_v[t, pl.ds(j * 16, 16)]
                m = jnp.where(v != 0, 1.0, 0.0).astype(jnp.float32)
                plsc.store_scatter(
                    tail_v.at[buf], [j * 16 + lanes, col0], m)
            line0 = t * NB + base
            pltpu.async_copy(
                tmp_v.at[buf],
                x_hbm.at[pl.ds(line0, STRIPE), pl.ds(0, EMB)],
                sem,
            )
            pltpu.async_copy(
                tail_v.at[buf],
                x_hbm.at[pl.ds(line0, STRIPE), pl.ds(EMB, LINE - EMB)],
                sem,
            )
        return carry

    lax.fori_loop(0, L // 2, gbody, 0)
    drain_pair(0, ssem0)
    drain_pair(1, ssem1)


def _sc_gather(f1p, f2p, table):
    mesh = plsc.VectorSubcoreMesh(core_axis_name="c", subcore_axis_name="s")
    f = pl.kernel(
        _sc_gather_body,
        out_type=jax.ShapeDtypeStruct((ROWS, LINE), jnp.float32),
        mesh=mesh,
        scratch_types=[
            pltpu.VMEM((STRIPE, LINE), jnp.int32),
            pltpu.VMEM((L, STRIPE), jnp.int32),
            pltpu.VMEM((2, STRIPE, EMB), jnp.float32),
            pltpu.VMEM((2, STRIPE, LINE - EMB), jnp.float32),
            pltpu.SemaphoreType.DMA,
            pltpu.SemaphoreType.DMA,
            pltpu.SemaphoreType.DMA,
        ],
        compiler_params=pltpu.CompilerParams(
            use_tc_tiling_on_sc=False, needs_layout_passes=False
        ),
    )
    return f(f1p, f2p, table)


def _tc_rnn_body(x_ref, w_ref, u_ref, b_ref, s1_ref, s2_ref, sim_ref, h_s):
    t = pl.program_id(0)

    @pl.when(t == 0)
    def _():
        h_s[...] = jnp.zeros_like(h_s)

    h = h_s[...]
    x = x_ref[0]
    xw = jnp.dot(x, w_ref[...], preferred_element_type=jnp.float32)
    hu = jnp.dot(h, u_ref[...], preferred_element_type=jnp.float32)
    h_new = jnp.tanh(xw + hu + b_ref[...])
    m = x[:, EMB:EMB + 1] != 0.0
    h = jnp.where(m, h_new, h)
    h_s[...] = h

    @pl.when(t == L - 1)
    def _():
        s1 = h[:B]
        s2 = h[B:]
        n1 = jnp.sqrt(jnp.sum(s1 * s1, axis=1, keepdims=True)) + 1e-12
        n2 = jnp.sqrt(jnp.sum(s2 * s2, axis=1, keepdims=True)) + 1e-12
        s1_ref[...] = s1
        s2_ref[...] = s2
        sim_ref[...] = jnp.sum(s1 * s2, axis=1, keepdims=True) / (n1 * n2)


def _tc_rnn(x, Wp, U, b):
    return pl.pallas_call(
        _tc_rnn_body,
        grid=(L,),
        in_specs=[
            pl.BlockSpec((1, NB, LINE), lambda t: (t, 0, 0)),
            pl.BlockSpec((LINE, FEAT), lambda t: (0, 0)),
            pl.BlockSpec((FEAT, FEAT), lambda t: (0, 0)),
            pl.BlockSpec((1, FEAT), lambda t: (0, 0)),
        ],
        out_specs=[
            pl.BlockSpec((B, FEAT), lambda t: (0, 0)),
            pl.BlockSpec((B, FEAT), lambda t: (0, 0)),
            pl.BlockSpec((B, 1), lambda t: (0, 0)),
        ],
        out_shape=[
            jax.ShapeDtypeStruct((B, FEAT), jnp.float32),
            jax.ShapeDtypeStruct((B, FEAT), jnp.float32),
            jax.ShapeDtypeStruct((B, 1), jnp.float32),
        ],
        scratch_shapes=[pltpu.VMEM((NB, FEAT), jnp.float32)],
    )(x, Wp, U, b)


@jax.jit
def kernel(funcname_1, funcname_2, emb_table, W, U, b):
    embt = jnp.transpose(emb_table)
    table = _repack(embt).reshape(VPAD, EMB)
    f1p = jnp.pad(funcname_1, ((0, 0), (0, LINE - L)))
    f2p = jnp.pad(funcname_2, ((0, 0), (0, LINE - L)))
    Wp = jnp.concatenate([W, jnp.zeros((LINE - EMB, FEAT), W.dtype)], axis=0)
    x = _sc_gather(f1p, f2p, table)
    x = x.reshape(L, NB, LINE)
    s1, s2, sim = _tc_rnn(x, Wp, U, b.reshape(1, FEAT))
    return (s1, s2, sim.reshape(B))


# speedup vs baseline: 1.3763x; 1.0224x over previous
import jax
import jax.numpy as jnp
from jax import lax
from jax.experimental import pallas as pl
from jax.experimental.pallas import tpu as pltpu
from jax.experimental.pallas import tpu_sc as plsc

B = 4096
L = 50
EMB = 64
FEAT = 64
LINE = 128
VOCAB1 = 1000001
NW = 32
NB = 2 * B
ROWS = NB * L
STRIPE = NB // NW
CHUNK = 128
CPS = STRIPE // CHUNK
TBLK = 16384
NTB = 31
HHALF = NTB * TBLK
VPAD = 2 * HHALF


def _repack_body(top_ref, bot_ref, out_ref):
    xt = jnp.transpose(top_ref[...], (1, 0))
    xb = jnp.transpose(bot_ref[...], (1, 0))
    out_ref[...] = jnp.concatenate([xt, xb], axis=1)


def _repack(embt):
    return pl.pallas_call(
        _repack_body,
        grid=(NTB,),
        in_specs=[
            pl.BlockSpec((EMB, TBLK), lambda g: (0, g)),
            pl.BlockSpec(
                (EMB, TBLK),
                lambda g: (0, jnp.minimum(g + NTB, (VOCAB1 - 1) // TBLK)),
            ),
        ],
        out_specs=pl.BlockSpec((TBLK, 2 * EMB), lambda g: (g, 0)),
        out_shape=jax.ShapeDtypeStruct((HHALF, 2 * EMB), jnp.float32),
    )(embt, embt)


def _make_sc_body(t0, nsteps):
  def _sc_gather_body(f1_hbm, f2_hbm, table_hbm, x_hbm,
                    idx_v, idxt_v, tmp_v, tail_v, gsem, ssem0, ssem1):
    c = lax.axis_index("c")
    s = lax.axis_index("s")
    w = s * 2 + c
    base = w * STRIPE

    @pl.when(w < NW // 2)
    def _():
        pltpu.sync_copy(f1_hbm.at[pl.ds(base, STRIPE)], idx_v)

    @pl.when(w >= NW // 2)
    def _():
        pltpu.sync_copy(f2_hbm.at[pl.ds(base - B, STRIPE)], idx_v)

    zeros16 = jnp.zeros((16,), jnp.float32)

    def zbody(r, carry):
        for k in range(4):
            tail_v[0, r, pl.ds(k * 16, 16)] = zeros16
            tail_v[1, r, pl.ds(k * 16, 16)] = zeros16
        return carry

    lax.fori_loop(0, STRIPE, zbody, 0)

    lanes = lax.iota(jnp.int32, 16)

    def tbody(t, carry):
        col = jnp.full((16,), t0 + t, dtype=jnp.int32)
        for j in range(STRIPE // 16):
            v = plsc.load_gather(idx_v, [j * 16 + lanes, col])
            ge = (v >= HHALF).astype(jnp.int32)
            v2 = 2 * v - (2 * HHALF) * ge + ge
            idxt_v[t, pl.ds(j * 16, 16)] = v2
        return carry

    lax.fori_loop(0, nsteps, tbody, 0)

    def drain_pair(buf, sem):
        pltpu.make_async_copy(
            tmp_v.at[buf],
            x_hbm.at[pl.ds(0, STRIPE), pl.ds(0, EMB)],
            sem,
        ).wait()
        pltpu.make_async_copy(
            tmp_v.at[buf],
            x_hbm.at[pl.ds(0, STRIPE), pl.ds(0, EMB)],
            sem,
        ).wait()

    sems = (ssem0, ssem1)
    col0 = jnp.full((16,), 0, dtype=jnp.int32)

    def gbody(g, carry):
        for buf in range(2):
            t = g * 2 + buf
            sem = sems[buf]

            @pl.when(g >= 1)
            def _():
                drain_pair(buf, sem)

            descs = []
            for j in range(CPS):
                d = pltpu.async_copy(
                    table_hbm.at[idxt_v.at[t, pl.ds(j * CHUNK, CHUNK)]],
                    tmp_v.at[buf, pl.ds(j * CHUNK, CHUNK)],
                    gsem,
                )
                descs.append(d)
            for d in descs:
                d.wait()
            for j in range(STRIPE // 16):
                v = idxt_v[t, pl.ds(j * 16, 16)]
                m = jnp.where(v != 0, 1.0, 0.0).astype(jnp.float32)
                plsc.store_scatter(
                    tail_v.at[buf], [j * 16 + lanes, col0], m)
            line0 = t * NB + base
            pltpu.async_copy(
                tmp_v.at[buf],
                x_hbm.at[pl.ds(line0, STRIPE), pl.ds(0, EMB)],
                sem,
            )
            pltpu.async_copy(
                tail_v.at[buf],
                x_hbm.at[pl.ds(line0, STRIPE), pl.ds(EMB, LINE - EMB)],
                sem,
            )
        return carry

    lax.fori_loop(0, nsteps // 2, gbody, 0)
    drain_pair(0, ssem0)
    drain_pair(1, ssem1)
  return _sc_gather_body


def _sc_gather(f1p, f2p, table, t0, nsteps):
    mesh = plsc.VectorSubcoreMesh(core_axis_name="c", subcore_axis_name="s")
    f = pl.kernel(
        _make_sc_body(t0, nsteps),
        out_type=jax.ShapeDtypeStruct((nsteps * NB, LINE), jnp.float32),
        mesh=mesh,
        scratch_types=[
            pltpu.VMEM((STRIPE, LINE), jnp.int32),
            pltpu.VMEM((nsteps, STRIPE), jnp.int32),
            pltpu.VMEM((2, STRIPE, EMB), jnp.float32),
            pltpu.VMEM((2, STRIPE, LINE - EMB), jnp.float32),
            pltpu.SemaphoreType.DMA,
            pltpu.SemaphoreType.DMA,
            pltpu.SemaphoreType.DMA,
        ],
        compiler_params=pltpu.CompilerParams(
            use_tc_tiling_on_sc=False, needs_layout_passes=False
        ),
    )
    return f(f1p, f2p, table)


def _make_tc_body(nsteps, final):
    def body(x_ref, w_ref, u_ref, b_ref, h0_ref, *refs):
        h_s = refs[-1]
        t = pl.program_id(0)

        @pl.when(t == 0)
        def _():
            h_s[...] = h0_ref[...]

        h = h_s[...]
        x = x_ref[0]
        xw = jnp.dot(x, w_ref[...], preferred_element_type=jnp.float32)
        hu = jnp.dot(h, u_ref[...], preferred_element_type=jnp.float32)
        h_new = jnp.tanh(xw + hu + b_ref[...])
        m = x[:, EMB:EMB + 1] != 0.0
        h = jnp.where(m, h_new, h)
        h_s[...] = h

        if final:
            s1_ref, s2_ref, sim_ref = refs[:3]

            @pl.when(t == nsteps - 1)
            def _():
                s1 = h[:B]
                s2 = h[B:]
                n1 = jnp.sqrt(jnp.sum(s1 * s1, axis=1, keepdims=True)) + 1e-12
                n2 = jnp.sqrt(jnp.sum(s2 * s2, axis=1, keepdims=True)) + 1e-12
                s1_ref[...] = s1
                s2_ref[...] = s2
                sim_ref[...] = jnp.sum(s1 * s2, axis=1, keepdims=True) / (n1 * n2)
        else:
            hout_ref = refs[0]

            @pl.when(t == nsteps - 1)
            def _():
                hout_ref[...] = h
    return body


def _tc_rnn_chunk(x, Wp, U, b, h0, nsteps, final):
    if final:
        out_shape = [
            jax.ShapeDtypeStruct((B, FEAT), jnp.float32),
            jax.ShapeDtypeStruct((B, FEAT), jnp.float32),
            jax.ShapeDtypeStruct((B, 1), jnp.float32),
        ]
        out_specs = [
            pl.BlockSpec((B, FEAT), lambda t: (0, 0)),
            pl.BlockSpec((B, FEAT), lambda t: (0, 0)),
            pl.BlockSpec((B, 1), lambda t: (0, 0)),
        ]
    else:
        out_shape = [jax.ShapeDtypeStruct((NB, FEAT), jnp.float32)]
        out_specs = [pl.BlockSpec((NB, FEAT), lambda t: (0, 0))]
    return pl.pallas_call(
        _make_tc_body(nsteps, final),
        grid=(nsteps,),
        in_specs=[
            pl.BlockSpec((1, NB, LINE), lambda t: (t, 0, 0)),
            pl.BlockSpec((LINE, FEAT), lambda t: (0, 0)),
            pl.BlockSpec((FEAT, FEAT), lambda t: (0, 0)),
            pl.BlockSpec((1, FEAT), lambda t: (0, 0)),
            pl.BlockSpec((NB, FEAT), lambda t: (0, 0)),
        ],
        out_specs=out_specs,
        out_shape=out_shape,
        scratch_shapes=[pltpu.VMEM((NB, FEAT), jnp.float32)],
    )(x, Wp, U, b, h0)


@jax.jit
def kernel(funcname_1, funcname_2, emb_table, W, U, b):
    embt = jnp.transpose(emb_table)
    table = _repack(embt).reshape(VPAD, EMB)
    f1p = jnp.pad(funcname_1, ((0, 0), (0, LINE - L)))
    f2p = jnp.pad(funcname_2, ((0, 0), (0, LINE - L)))
    Wp = jnp.concatenate([W, jnp.zeros((LINE - EMB, FEAT), W.dtype)], axis=0)
    T0 = 26
    x0 = _sc_gather(f1p, f2p, table, 0, T0).reshape(T0, NB, LINE)
    x1 = _sc_gather(f1p, f2p, table, T0, L - T0).reshape(L - T0, NB, LINE)
    h0 = jnp.zeros((NB, FEAT), jnp.float32)
    bb = b.reshape(1, FEAT)
    h1 = _tc_rnn_chunk(x0, Wp, U, bb, h0, T0, final=False)[0]
    s1, s2, sim = _tc_rnn_chunk(x1, Wp, U, bb, h1, L - T0, final=True)
    return (s1, s2, sim.reshape(B))
